# baseline (speedup 1.0000x reference)
import jax
import jax.numpy as jnp
from jax.experimental import pallas as pl

B = 1024
SIZE = 26
D = 4096
NUM_LEVELS = 100
D_TILE = 512
K = SIZE * 128


def _onehot_kernel(x_ref, g_ref):
    idx = jnp.clip(jnp.floor(x_ref[...] * NUM_LEVELS), 0, NUM_LEVELS - 1)
    idx = idx.astype(jnp.int32)
    iota = jax.lax.broadcasted_iota(jnp.int32, (1, 128), 1)
    hots = []
    for s in range(SIZE):
        hots.append((idx[:, s][:, None] == iota).astype(jnp.bfloat16))
    g_ref[...] = jnp.concatenate(hots, axis=1)


def _matmul_kernel(g_ref, pos_ref, lev_ref, out_ref):
    lev = jnp.concatenate(
        [lev_ref[...].astype(jnp.bfloat16),
         jnp.zeros((128 - NUM_LEVELS, D_TILE), jnp.bfloat16)], axis=0)
    tabs = []
    for s in range(SIZE):
        w = pos_ref[s, :][None, :].astype(jnp.bfloat16)
        tabs.append(lev + w - 2.0 * lev * w)
    table = jnp.concatenate(tabs, axis=0)
    out_ref[...] = jnp.dot(g_ref[...], table,
                           preferred_element_type=jnp.float32)


@jax.jit
def kernel(x, position, levels):
    g = pl.pallas_call(
        _onehot_kernel,
        out_shape=jax.ShapeDtypeStruct((B, K), jnp.bfloat16),
    )(x)
    return pl.pallas_call(
        _matmul_kernel,
        grid=(D // D_TILE,),
        in_specs=[
            pl.BlockSpec((B, K), lambda j: (0, 0)),
            pl.BlockSpec((SIZE, D_TILE), lambda j: (0, j)),
            pl.BlockSpec((NUM_LEVELS, D_TILE), lambda j: (0, j)),
        ],
        out_specs=pl.BlockSpec((B, D_TILE), lambda j: (0, j)),
        out_shape=jax.ShapeDtypeStruct((B, D), jnp.float32),
    )(g, position, levels)
